# baseline (speedup 1.0000x reference)
import functools

import jax
import jax.numpy as jnp
from jax import lax
from jax.experimental import pallas as pl
from jax.experimental.pallas import tpu as pltpu
from jax.experimental.pallas import tpu_sc as plsc

B, L, D = 16384, 200, 10
NC, NS, LANES = 2, 16, 16
NW = NC * NS
BW = B // NW
NL = 8
NWAVES = L // NL
GROUPS = BW // LANES
NPAIRS = (NWAVES - 1) // 2


def _make_emb():
    mesh = plsc.VectorSubcoreMesh(core_axis_name="c", subcore_axis_name="s")

    @functools.partial(
        pl.kernel,
        mesh=mesh,
        out_type=jax.ShapeDtypeStruct((D, L, B), jnp.float32),
        scratch_types=[
            pltpu.VMEM((D, 100), jnp.float32),
            pltpu.VMEM((2, NL, BW), jnp.int32),
            pltpu.VMEM((2, D, NL, BW), jnp.float32),
            pltpu.SemaphoreType.DMA,
            pltpu.SemaphoreType.DMA,
            pltpu.SemaphoreType.DMA,
            pltpu.SemaphoreType.DMA,
        ],
        compiler_params=pltpu.CompilerParams(needs_layout_passes=False),
    )
    def emb(ids_hbm, tabt_hbm, out_hbm, tabt_v, idx_v, rows_v,
            in0, in1, out0, out1):
        wid = lax.axis_index("s") * NC + lax.axis_index("c")
        b0 = wid * BW
        pltpu.sync_copy(tabt_hbm, tabt_v)
        dsplat = [jnp.full((LANES,), d, jnp.int32) for d in range(D)]
        in_sems = [in0, in1]
        out_sems = [out0, out1]

        def in_dma(slot, w):
            return pltpu.make_async_copy(
                ids_hbm.at[pl.ds(w * NL, NL), pl.ds(b0, BW)],
                idx_v.at[slot], in_sems[slot])

        def out_dma_start(slot, w):
            for d in range(D):
                pltpu.make_async_copy(
                    rows_v.at[slot, d],
                    out_hbm.at[d, pl.ds(w * NL, NL), pl.ds(b0, BW)],
                    out_sems[slot]).start()

        def out_dma_wait(slot, w):
            for d in range(D):
                pltpu.make_async_copy(
                    rows_v.at[slot, d],
                    out_hbm.at[d, pl.ds(w * NL, NL), pl.ds(b0, BW)],
                    out_sems[slot]).wait()

        def compute(slot):
            @functools.partial(plsc.parallel_loop, 0, GROUPS, unroll=2)
            def _(g):
                off = g * LANES
                for l in range(NL):
                    ids = idx_v[slot, l, pl.ds(off, LANES)]
                    for d in range(D):
                        col = plsc.load_gather(tabt_v, [dsplat[d], ids])
                        rows_v[slot, d, l, pl.ds(off, LANES)] = col

        def wave(slot, w, i, prefetch_slot, prefetch_w):
            in_dma(prefetch_slot, prefetch_w).start()
            in_dma(slot, w).wait()

            @pl.when(i > 0)
            def _():
                out_dma_wait(slot, w - 2)

            compute(slot)
            out_dma_start(slot, w)

        in_dma(0, 0).start()

        def pair_body(i, carry):
            w0 = 2 * i
            wave(0, w0, i, 1, w0 + 1)
            wave(1, w0 + 1, i, 0, w0 + 2)
            return carry

        lax.fori_loop(0, NPAIRS, pair_body, 0)

        last = NWAVES - 1
        in_dma(0, last).wait()
        out_dma_wait(0, last - 2)
        compute(0)
        out_dma_start(0, last)
        out_dma_wait(1, last - 1)
        out_dma_wait(0, last)

    return emb


_emb = _make_emb()


def kernel(input_ids, table):
    idsT = input_ids.T.astype(jnp.int32)
    tabT = table.T
    out = _emb(idsT, tabT)
    return out.transpose(2, 1, 0)
